# Initial kernel scaffold; baseline (speedup 1.0000x reference)
import jax
import jax.numpy as jnp
from jax.experimental import pallas as pl


def kernel(pred, target):
    raise NotImplementedError("write your pallas kernel here")


# speedup vs baseline: 1.3914x; 1.3914x over previous
import functools

import numpy as np
import jax
import jax.numpy as jnp
from jax.experimental import pallas as pl
from jax.experimental.pallas import tpu as pltpu

_BINS = 10
_EPS = 1e-8


def _edges_f32():
    e = np.arange(_BINS + 1, dtype=np.float32) / np.float32(_BINS)
    e[-1] = np.float32(e[-1] + np.float32(1e-6))
    return [float(v) for v in e]


def _tc_all_body(tgt_ref, pred_ref, out_ref, cnt_ref, lsum_ref):
    i = pl.program_id(0)
    nb = pl.num_programs(0)

    @pl.when(i == 0)
    def _init():
        for b in range(_BINS):
            cnt_ref[b] = 0.0
            lsum_ref[b] = 0.0

    x = pred_ref[...]
    r, c = x.shape
    t = tgt_ref[0, 0, :]
    col = jax.lax.broadcasted_iota(jnp.int32, (r, c), 1)
    onehot = col == t[:, None]
    e = jnp.exp(x)
    s = jnp.sum(e, axis=1)
    m = jnp.sum(jnp.where(onehot, x, 0.0), axis=1)
    p = jnp.exp(m) / s
    g = 1.0 - p
    loss = -m + jnp.log(s + _EPS)

    edges = _edges_f32()
    nge = jnp.zeros((r,), jnp.int32)
    for ev in edges:
        nge = nge + (g >= ev).astype(jnp.int32)
    bin_idx = jnp.clip(nge - 1, 0, _BINS - 1)

    for b in range(_BINS):
        mb = bin_idx == b
        cnt_ref[b] += jnp.sum(mb.astype(jnp.float32))
        lsum_ref[b] += jnp.sum(jnp.where(mb, loss, 0.0))

    @pl.when(i == nb - 1)
    def _fin():
        n = 0.0
        tot = 0.0
        for b in range(_BINS):
            cb = cnt_ref[b]
            nz = cb > 0.0
            n += jnp.where(nz, 1.0, 0.0)
            tot += jnp.where(nz, lsum_ref[b] / jnp.maximum(cb, 1.0), 0.0)
        out_ref[0, 0] = tot / jnp.maximum(n, 1.0)


def _ghmc_tc(pred, target, row_block=512, interpret=False):
    bsz, csz = pred.shape
    nb = bsz // row_block
    tgt3 = target.reshape(nb, 1, row_block)
    out = pl.pallas_call(
        _tc_all_body,
        grid=(nb,),
        in_specs=[
            pl.BlockSpec((1, 1, row_block), lambda i: (i, 0, 0)),
            pl.BlockSpec((row_block, csz), lambda i: (i, 0)),
        ],
        out_specs=pl.BlockSpec(
            (1, 1), lambda i: (0, 0), memory_space=pltpu.SMEM
        ),
        out_shape=jax.ShapeDtypeStruct((1, 1), jnp.float32),
        scratch_shapes=[
            pltpu.SMEM((_BINS,), jnp.float32),
            pltpu.SMEM((_BINS,), jnp.float32),
        ],
        interpret=interpret,
    )(tgt3, pred)
    return out[0, 0]


def kernel(pred, target):
    return _ghmc_tc(pred, target)
